# baseline (speedup 1.0000x reference)
import functools

import jax
import jax.numpy as jnp
from jax import lax
from jax.experimental import pallas as pl
from jax.experimental.pallas import tpu as pltpu
from jax.experimental.pallas import tpu_sc as plsc

NC, NS, L = 2, 16, 16
NT = NC * NS
P_PAD = 1_048_576
ROWS = P_PAD // 128
NB = 40_000
NB_PAD = 40_064
NF = 16_896
C = 128
CQ = 32
NQ = 4

A_WIN_ROWS = 8
A_WINS = 32
B_WIN_ROWS = 1
B_WINS = 512
B_WPTS = B_WIN_ROWS * 128
B_TROWS = B_WINS * B_WIN_ROWS


def _mesh():
    return plsc.VectorSubcoreMesh(
        core_axis_name="c", subcore_axis_name="s",
        num_cores=NC, num_subcores=NS)


@functools.partial(
    pl.kernel,
    out_type=jax.ShapeDtypeStruct((ROWS, 128), jnp.float32),
    mesh=_mesh(),
    compiler_params=pltpu.CompilerParams(needs_layout_passes=False, use_tc_tiling_on_sc=False),
    scratch_types=[
        pltpu.VMEM((A_WIN_ROWS, 128), jnp.int32),
        pltpu.VMEM((A_WIN_ROWS, 128), jnp.float32),
        pltpu.SemaphoreType.DMA,
    ],
)
def _gather_depth(depth_hbm, rd_hbm, gd_hbm, idx_v, val_v, sem):
    wid = lax.axis_index("s") * NC + lax.axis_index("c")
    row0 = wid * (A_WINS * A_WIN_ROWS)

    def win(w, _):
        r0 = row0 + w * A_WIN_ROWS
        pltpu.sync_copy(rd_hbm.at[pl.ds(r0, A_WIN_ROWS)], idx_v)
        cps = [pltpu.async_copy(depth_hbm.at[idx_v.at[j]], val_v.at[j], sem)
               for j in range(A_WIN_ROWS)]
        for cp in cps:
            cp.wait()
        pltpu.sync_copy(val_v, gd_hbm.at[pl.ds(r0, A_WIN_ROWS)])
        return 0

    lax.fori_loop(0, A_WINS, win, 0)


@functools.partial(
    pl.kernel,
    out_type=jax.ShapeDtypeStruct((NQ, NB_PAD, CQ), jnp.float32),
    mesh=_mesh(),
    compiler_params=pltpu.CompilerParams(needs_layout_passes=False, use_tc_tiling_on_sc=False),
    scratch_types=[
        pltpu.VMEM_SHARED((NB_PAD, CQ), jnp.float32),
        pltpu.VMEM_SHARED((NF, CQ), jnp.float32),
        pltpu.VMEM((B_WIN_ROWS, 128), jnp.int32),
        pltpu.VMEM((B_WIN_ROWS, 128), jnp.int32),
        pltpu.VMEM((B_WIN_ROWS, 128), jnp.int32),
        pltpu.VMEM((B_WIN_ROWS, 128), jnp.int32),
        pltpu.VMEM((B_WPTS,), jnp.float32),
        pltpu.VMEM((B_WPTS,), jnp.float32),
        pltpu.VMEM((B_WPTS, CQ), jnp.float32),
        pltpu.VMEM((B_WPTS, CQ), jnp.float32),
        pltpu.SemaphoreType.DMA,
        pltpu.SemaphoreType.DMA,
        pltpu.SemaphoreType.DMA,
        pltpu.SemaphoreType.DMA,
    ],
)
def _pool(ftab_hbm, rf_hbm, rb_hbm, gd_hbm, zeros_hbm, out_hbm,
          acc, ftab_s, rf0, rf1, rb0, rb1, gd0, gd1, rows0, rows1,
          gsem0, gsem1, ssem0, ssem1):
    c = lax.axis_index("c")
    s = lax.axis_index("s")
    zrows = NB_PAD // NS
    base = s * B_TROWS
    last = B_TROWS - B_WIN_ROWS

    def load_idx(r0, rf_v, rb_v, gd_v):
        pltpu.sync_copy(rf_hbm.at[pl.ds(base + r0, B_WIN_ROWS)], rf_v)
        pltpu.sync_copy(rb_hbm.at[pl.ds(base + r0, B_WIN_ROWS)], rb_v)
        pltpu.sync_copy(gd_hbm.at[pl.ds((base + r0) * 128, B_WPTS)], gd_v)

    frows = NF // NS

    def fire_gather(q, rf_v, rows_v, gsem):
        return [pltpu.async_copy(ftab_s.at[rf_v.at[j]],
                                 rows_v.at[pl.ds(j * 128, 128)], gsem)
                for j in range(B_WIN_ROWS)]

    def fire_scatter(rb_v, rows_v, ssem):
        return [pltpu.async_copy(rows_v.at[pl.ds(j * 128, 128)],
                                 acc.at[rb_v.at[j]], ssem, add=True)
                for j in range(B_WIN_ROWS)]

    def mul_win(rows_v, gd_v):
        def mul16(b, _):
            p0 = b * L
            d16 = gd_v[pl.ds(p0, L)]
            for l in range(L):
                dl = lax.broadcast_in_dim(d16[l], (L,), ())
                i = p0 + l
                rows_v[i, pl.ds(0, L)] = rows_v[i, pl.ds(0, L)] * dl
                rows_v[i, pl.ds(L, L)] = rows_v[i, pl.ds(L, L)] * dl
            return 0
        lax.fori_loop(0, B_WPTS // L, mul16, 0)

    for rnd in range(NQ // NC):
        q = c * (NQ // NC) + rnd
        pltpu.sync_copy(zeros_hbm, acc.at[pl.ds(s * zrows, zrows)])
        pltpu.sync_copy(ftab_hbm.at[q].at[pl.ds(s * frows, frows)],
                        ftab_s.at[pl.ds(s * frows, frows)])
        plsc.subcore_barrier()

        load_idx(0, rf0, rb0, gd0)
        for h in fire_gather(q, rf0, rows0, gsem0):
            h.wait()

        def pair(g, _):
            w0r = 2 * g * B_WIN_ROWS
            load_idx(w0r + B_WIN_ROWS, rf1, rb1, gd1)
            hg1 = fire_gather(q, rf1, rows1, gsem1)
            mul_win(rows0, gd0)
            hs0 = fire_scatter(rb0, rows0, ssem0)
            for h in hg1:
                h.wait()
            for h in hs0:
                h.wait()
            load_idx(jnp.minimum(w0r + 2 * B_WIN_ROWS, last), rf0, rb0, gd0)
            hg0 = fire_gather(q, rf0, rows0, gsem0)
            mul_win(rows1, gd1)
            hs1 = fire_scatter(rb1, rows1, ssem1)
            for h in hg0:
                h.wait()
            for h in hs1:
                h.wait()
            return 0

        lax.fori_loop(0, B_WINS // 2, pair, 0)
        plsc.subcore_barrier()
        pltpu.sync_copy(acc.at[pl.ds(s * zrows, zrows)],
                        out_hbm.at[q].at[pl.ds(s * zrows, zrows)])
        plsc.subcore_barrier()


def kernel(ranks_depth, ranks_feat, ranks_bev, n_points, depth, feat):
    P = ranks_depth.shape[0]
    depth_flat = depth.reshape(-1)
    dsz = depth_flat.shape[0]
    ftab = feat.reshape(NF, NQ, CQ).transpose(1, 0, 2)

    pad = P_PAD - P
    ar = jnp.arange(pad, dtype=jnp.int32)
    rd = jnp.concatenate([ranks_depth, ar % dsz]).reshape(ROWS, 128)
    rf = jnp.concatenate([ranks_feat, ar % NF]).reshape(ROWS, 128)
    rb = jnp.concatenate([ranks_bev, ar % NB]).reshape(ROWS, 128)

    gd = _gather_depth(depth_flat, rd).reshape(-1)
    valid = jnp.arange(P_PAD, dtype=jnp.int32) < n_points
    gd = jnp.where(valid, gd, 0.0)

    zeros = jnp.zeros((NB_PAD // NS, CQ), jnp.float32)
    out4 = _pool(ftab, rf, rb, gd, zeros)
    out = out4[:, :NB, :].transpose(1, 0, 2).reshape(NB, C)
    return out.reshape(1, 1, 200, 200, C)


# speedup vs baseline: 2.0706x; 2.0706x over previous
import functools

import jax
import jax.numpy as jnp
from jax import lax
from jax.experimental import pallas as pl
from jax.experimental.pallas import tpu as pltpu
from jax.experimental.pallas import tpu_sc as plsc

NC, NS, L = 2, 16, 16
NT = NC * NS
P_PAD = 1_048_576
ROWS = P_PAD // 128
NB = 40_000
NB_PAD = 40_064
NF = 16_896
C = 128
CQ = 32
NQ = 4

A_WIN_ROWS = 8
A_WINS = 32
B_CHUNK_ROWS = 8
B_CHUNKS = 64
B_SUB = 128
B_TROWS = B_CHUNKS * B_CHUNK_ROWS


def _mesh():
    return plsc.VectorSubcoreMesh(
        core_axis_name="c", subcore_axis_name="s",
        num_cores=NC, num_subcores=NS)


@functools.partial(
    pl.kernel,
    out_type=jax.ShapeDtypeStruct((ROWS, 128), jnp.float32),
    mesh=_mesh(),
    compiler_params=pltpu.CompilerParams(needs_layout_passes=False, use_tc_tiling_on_sc=False),
    scratch_types=[
        pltpu.VMEM((A_WIN_ROWS, 128), jnp.int32),
        pltpu.VMEM((A_WIN_ROWS, 128), jnp.float32),
        pltpu.SemaphoreType.DMA,
    ],
)
def _gather_depth(depth_hbm, rd_hbm, gd_hbm, idx_v, val_v, sem):
    wid = lax.axis_index("s") * NC + lax.axis_index("c")
    row0 = wid * (A_WINS * A_WIN_ROWS)

    def win(w, _):
        r0 = row0 + w * A_WIN_ROWS
        pltpu.sync_copy(rd_hbm.at[pl.ds(r0, A_WIN_ROWS)], idx_v)
        cps = [pltpu.async_copy(depth_hbm.at[idx_v.at[j]], val_v.at[j], sem)
               for j in range(A_WIN_ROWS)]
        for cp in cps:
            cp.wait()
        pltpu.sync_copy(val_v, gd_hbm.at[pl.ds(r0, A_WIN_ROWS)])
        return 0

    lax.fori_loop(0, A_WINS, win, 0)


@functools.partial(
    pl.kernel,
    out_type=jax.ShapeDtypeStruct((NQ, NB_PAD, CQ), jnp.float32),
    mesh=_mesh(),
    compiler_params=pltpu.CompilerParams(needs_layout_passes=False, use_tc_tiling_on_sc=False),
    scratch_types=[
        pltpu.VMEM_SHARED((NB_PAD, CQ), jnp.float32),
        pltpu.VMEM_SHARED((NF, CQ), jnp.float32),
        pltpu.VMEM((B_CHUNK_ROWS, 128), jnp.int32),
        pltpu.VMEM((B_CHUNK_ROWS, 128), jnp.int32),
        pltpu.VMEM((B_CHUNK_ROWS * 128,), jnp.float32),
        pltpu.VMEM((B_SUB, CQ), jnp.float32),
        pltpu.VMEM((B_SUB, CQ), jnp.float32),
        pltpu.VMEM((B_SUB, CQ), jnp.float32),
        pltpu.SemaphoreType.DMA,
        pltpu.SemaphoreType.DMA,
        pltpu.SemaphoreType.DMA,
        pltpu.SemaphoreType.DMA,
        pltpu.SemaphoreType.DMA,
        pltpu.SemaphoreType.DMA,
    ],
)
def _pool(ftab_hbm, rf_hbm, rb_hbm, gd_hbm, zeros_hbm, out_hbm,
          acc, ftab_s, rfC, rbC, gdC, rows0, rows1, rows2,
          gsem0, gsem1, gsem2, ssem0, ssem1, ssem2):
    c = lax.axis_index("c")
    s = lax.axis_index("s")
    zrows = NB_PAD // NS
    frows = NF // NS
    base = s * B_TROWS
    rows = (rows0, rows1, rows2)
    gsems = (gsem0, gsem1, gsem2)
    ssems = (ssem0, ssem1, ssem2)

    def mul_sub(rows_v, j):
        def mul16(bb, _):
            p0 = bb * L
            d16 = gdC[pl.ds(j * B_SUB + p0, L)]
            for l in range(L):
                dl = lax.broadcast_in_dim(d16[l], (L,), ())
                i = p0 + l
                rows_v[i, pl.ds(0, L)] = rows_v[i, pl.ds(0, L)] * dl
                rows_v[i, pl.ds(L, L)] = rows_v[i, pl.ds(L, L)] * dl
            return 0
        lax.fori_loop(0, B_SUB // L, mul16, 0)

    for rnd in range(NQ // NC):
        q = c * (NQ // NC) + rnd
        pltpu.sync_copy(zeros_hbm, acc.at[pl.ds(s * zrows, zrows)])
        pltpu.sync_copy(ftab_hbm.at[q].at[pl.ds(s * frows, frows)],
                        ftab_s.at[pl.ds(s * frows, frows)])
        plsc.subcore_barrier()

        def chunk(kk, _):
            r0 = base + kk * B_CHUNK_ROWS
            pltpu.sync_copy(rf_hbm.at[pl.ds(r0, B_CHUNK_ROWS)], rfC)
            pltpu.sync_copy(rb_hbm.at[pl.ds(r0, B_CHUNK_ROWS)], rbC)
            pltpu.sync_copy(gd_hbm.at[pl.ds(r0 * 128, B_CHUNK_ROWS * 128)],
                            gdC)
            hg = {}
            hs = {}
            hg[0] = pltpu.async_copy(ftab_s.at[rfC.at[0]], rows[0], gsems[0])
            for j in range(B_CHUNK_ROWS):
                b = j % 3
                if j >= 2:
                    hs[j - 2].wait()
                if j + 1 < B_CHUNK_ROWS:
                    nb = (j + 1) % 3
                    hg[j + 1] = pltpu.async_copy(
                        ftab_s.at[rfC.at[j + 1]], rows[nb], gsems[nb])
                hg[j].wait()
                mul_sub(rows[b], j)
                hs[j] = pltpu.async_copy(rows[b], acc.at[rbC.at[j]],
                                         ssems[b], add=True)
            hs[B_CHUNK_ROWS - 2].wait()
            hs[B_CHUNK_ROWS - 1].wait()
            return 0

        lax.fori_loop(0, B_CHUNKS, chunk, 0)
        plsc.subcore_barrier()
        pltpu.sync_copy(acc.at[pl.ds(s * zrows, zrows)],
                        out_hbm.at[q].at[pl.ds(s * zrows, zrows)])
        plsc.subcore_barrier()


def kernel(ranks_depth, ranks_feat, ranks_bev, n_points, depth, feat):
    P = ranks_depth.shape[0]
    depth_flat = depth.reshape(-1)
    dsz = depth_flat.shape[0]
    ftab = feat.reshape(NF, NQ, CQ).transpose(1, 0, 2)

    pad = P_PAD - P
    ar = jnp.arange(pad, dtype=jnp.int32)
    rd = jnp.concatenate([ranks_depth, ar % dsz]).reshape(ROWS, 128)
    rf = jnp.concatenate([ranks_feat, ar % NF]).reshape(ROWS, 128)
    rb = jnp.concatenate([ranks_bev, ar % NB]).reshape(ROWS, 128)

    gd = _gather_depth(depth_flat, rd).reshape(-1)
    valid = jnp.arange(P_PAD, dtype=jnp.int32) < n_points
    gd = jnp.where(valid, gd, 0.0)

    zeros = jnp.zeros((NB_PAD // NS, CQ), jnp.float32)
    out4 = _pool(ftab, rf, rb, gd, zeros)
    out = out4[:, :NB, :].transpose(1, 0, 2).reshape(NB, C)
    return out.reshape(1, 1, 200, 200, C)
